# baseline (speedup 1.0000x reference)
import functools

import jax
import jax.numpy as jnp
from jax import lax
from jax.experimental import pallas as pl
from jax.experimental.pallas import tpu as pltpu
from jax.experimental.pallas import tpu_sc as plsc

B, P, D = 64, 8192, 128
K = 64
R, C = 64, 128
NEG_INF = float("-inf")
POS_INF = float("inf")


def _allreduce_rows(x, op):
    x = op(x[0:32], x[32:64])
    x = op(x[0:16], x[16:32])
    x = op(x[0:8], x[8:16])
    x = op(x, pltpu.roll(x, 4, 0))
    x = op(x, pltpu.roll(x, 2, 0))
    x = op(x, pltpu.roll(x, 1, 0))
    return x[0:1]


def _allreduce_lanes(x, op):
    for sh in (64, 32, 16, 8, 4, 2, 1):
        x = op(x, pltpu.roll(x, sh, 1))
    return x


def _sim_topk_body(ln_ref, pn_ref, ti_ref, bi_ref):
    lb = ln_ref[0].astype(jnp.bfloat16)
    pb = pn_ref[0].astype(jnp.bfloat16)
    s = lax.dot_general(pb, lb, (((1,), (1,)), ((), ())),
                        preferred_element_type=jnp.float32)
    sm = s.reshape(R, C)

    lane = lax.broadcasted_iota(jnp.int32, (R, C), 1)
    row = lax.broadcasted_iota(jnp.int32, (R, C), 0)
    flat = row * C + lane
    out_iota = lax.broadcasted_iota(jnp.int32, (1, K), 1)
    big = jnp.full((R, C), P, jnp.int32)

    def extract(w, pad):
        m = _allreduce_lanes(_allreduce_rows(w, jnp.maximum), jnp.maximum)
        cand = jnp.where(w == m, flat, big)
        pos = _allreduce_lanes(_allreduce_rows(cand, jnp.minimum), jnp.minimum)
        w = jnp.where(flat == pos, pad, w)
        return w, pos[:, 0:K]

    padt = jnp.full((R, C), NEG_INF, jnp.float32)

    def rnd(r, carry):
        wt, wb, ti, bi = carry
        wt, post = extract(wt, padt)
        ti = jnp.where(out_iota == r, post, ti)
        wb, posb = extract(wb, padt)
        bi = jnp.where(out_iota == r, posb, bi)
        return wt, wb, ti, bi

    init = (sm, -sm, jnp.zeros((1, K), jnp.int32), jnp.zeros((1, K), jnp.int32))
    _, _, ti, bi = lax.fori_loop(0, K, rnd, init)
    ti_ref[...] = ti.reshape(1, 1, K)
    bi_ref[...] = bi.reshape(1, 1, K)


def _compute_topk_idx(ln, pn):
    return pl.pallas_call(
        _sim_topk_body,
        grid=(B,),
        in_specs=[
            pl.BlockSpec((1, P, D), lambda b: (b, 0, 0)),
            pl.BlockSpec((1, 1, D), lambda b: (b, 0, 0)),
        ],
        out_specs=[
            pl.BlockSpec((1, 1, K), lambda b: (b, 0, 0)),
            pl.BlockSpec((1, 1, K), lambda b: (b, 0, 0)),
        ],
        out_shape=[
            jax.ShapeDtypeStruct((B, 1, K), jnp.int32),
            jax.ShapeDtypeStruct((B, 1, K), jnp.int32),
        ],
    )(ln, pn)


def _make_gather_kernel():
    info = plsc.get_sparse_core_info()
    nc = info.num_cores
    mesh = plsc.VectorSubcoreMesh(core_axis_name="c", subcore_axis_name="s")

    @functools.partial(
        pl.kernel,
        out_type=(
            jax.ShapeDtypeStruct((B, K, D), jnp.float32),
            jax.ShapeDtypeStruct((B, K, D), jnp.float32),
        ),
        mesh=mesh,
        scratch_types=[
            pltpu.VMEM((K,), jnp.int32),
            pltpu.VMEM((K,), jnp.int32),
            pltpu.VMEM((16,), jnp.int32),
            pltpu.VMEM((K, D), jnp.float32),
            pltpu.SemaphoreType.DMA,
        ],
    )
    def gather_kernel(ti_hbm, bi_hbm, base_hbm, feat_hbm, tf_hbm, bf_hbm,
                      idxv, gidx, bv, rows, sem):
        w = lax.axis_index("s") * nc + lax.axis_index("c")
        for bi_ in range(B // 32):
            b = w * (B // 32) + bi_
            pltpu.sync_copy(base_hbm.at[b], bv)
            base16 = bv[...]
            for side in range(2):
                src = ti_hbm if side == 0 else bi_hbm
                dst = tf_hbm if side == 0 else bf_hbm
                pltpu.sync_copy(src.at[b], idxv)
                for j in range(K // 16):
                    gidx[pl.ds(j * 16, 16)] = idxv[pl.ds(j * 16, 16)] + base16
                pltpu.async_copy(feat_hbm.at[gidx], rows, sem).wait()
                pltpu.sync_copy(rows, dst.at[b])

    return gather_kernel


_gather_kernel = _make_gather_kernel()


def kernel(local_features, prototypes):
    ln = local_features / (jnp.linalg.norm(local_features, axis=-1, keepdims=True) + 1e-08)
    pn = prototypes / (jnp.linalg.norm(prototypes, axis=-1, keepdims=True) + 1e-08)
    ti3, bi3 = _compute_topk_idx(ln, pn)
    ti = ti3.reshape(B, K)
    bi = bi3.reshape(B, K)
    bases = jnp.broadcast_to((jnp.arange(B, dtype=jnp.int32) * P)[:, None], (B, 16))
    featrows = local_features.reshape(B * P, D)
    tf, bf = _gather_kernel(ti, bi, bases, featrows)
    return tf, bf, ti, bi


# speedup vs baseline: 3.7200x; 3.7200x over previous
import functools

import jax
import jax.numpy as jnp
from jax import lax
from jax.experimental import pallas as pl
from jax.experimental.pallas import tpu as pltpu
from jax.experimental.pallas import tpu_sc as plsc

B, P, D = 64, 8192, 128
K = 64
R, C = 64, 128
NEG_INF = float("-inf")
POS_INF = float("inf")


def _allreduce_rows(x, op):
    x = op(x[0:32], x[32:64])
    x = op(x[0:16], x[16:32])
    x = op(x[0:8], x[8:16])
    x = op(x, pltpu.roll(x, 4, 0))
    x = op(x, pltpu.roll(x, 2, 0))
    x = op(x, pltpu.roll(x, 1, 0))
    return x[0:1]


def _allreduce_lanes(x, op):
    for sh in (64, 32, 16, 8, 4, 2, 1):
        x = op(x, pltpu.roll(x, sh, 1))
    return x


def _dotH(a, b, dims):
    return lax.dot_general(a, b, (dims, ((), ())),
                           precision=lax.Precision.HIGHEST,
                           preferred_element_type=jnp.float32)


def _allreduce_col(x, op):
    n = x.shape[0]
    while n > 8:
        x = op(x[0 : n // 2], x[n // 2 : n])
        n //= 2
    x = op(x, pltpu.roll(x, 4, 0))
    x = op(x, pltpu.roll(x, 2, 0))
    x = op(x, pltpu.roll(x, 1, 0))
    return x[0:1]


def _sim_topk_body(ln_ref, pn_ref, ti_ref, bi_ref):
    l = ln_ref[0]
    ss = jnp.sum(l * l, axis=-1, keepdims=True)
    nrm = jnp.where(ss == 0.0, 0.0, ss * lax.rsqrt(ss)) + 1e-8
    lb = (l * (1.0 / nrm)).astype(jnp.bfloat16)
    pb = pn_ref[0].astype(jnp.bfloat16)
    s = lax.dot_general(pb, lb, (((1,), (1,)), ((), ())),
                        preferred_element_type=jnp.float32)
    sm = s.reshape(R, C)

    f32 = jnp.float32
    lane = lax.broadcasted_iota(jnp.int32, (R, C), 1)
    row = lax.broadcasted_iota(jnp.int32, (R, C), 0)
    flat = row * C + lane
    out_iota = lax.broadcasted_iota(jnp.int32, (1, K), 1)
    out_iota_f = out_iota.astype(f32)
    big = jnp.full((R, C), P, jnp.int32)

    eye = jnp.where(
        lax.broadcasted_iota(jnp.int32, (C, C), 0)
        == lax.broadcasted_iota(jnp.int32, (C, C), 1), 1.0, 0.0).astype(f32)
    lt_ll = jnp.where(
        lax.broadcasted_iota(jnp.int32, (C, C), 0)
        < lax.broadcasted_iota(jnp.int32, (C, C), 1), 1.0, 0.0).astype(f32)
    lt_rr = jnp.where(
        lax.broadcasted_iota(jnp.int32, (R, R), 1)
        < lax.broadcasted_iota(jnp.int32, (R, R), 0), 1.0, 0.0).astype(f32)
    ones_col = jnp.full((C, 1), 1.0, f32)
    idx128 = lax.broadcasted_iota(jnp.int32, (128, 128), 0)
    lane128 = lax.broadcasted_iota(jnp.int32, (128, 128), 1)
    flat_row = lax.broadcasted_iota(jnp.int32, (1, P), 1).astype(f32)
    ones_row = jnp.full((1, P), 1.0, f32)
    slot_iota = lax.broadcasted_iota(jnp.int32, (128, P), 0).astype(f32)

    def fast_select(smx, s_row):
        cm = _allreduce_rows(smx, jnp.maximum)
        cmT = _dotH(eye, cm, ((1,), (1,)))
        b0 = jnp.where((cm > cmT) | ((cm == cmT) & (lane128 < idx128)),
                       1.0, 0.0).astype(f32)
        rank0 = _dotH(b0, ones_col, ((1,), (0,)))
        t0 = _allreduce_col(
            jnp.where(rank0 == (K - 1.0), cmT, NEG_INF), jnp.maximum)
        mask = smx >= t0
        mf = jnp.where(mask, 1.0, 0.0).astype(f32)
        rowcnt = _dotH(mf, ones_col, ((1,), (0,)))
        tot = _allreduce_col(rowcnt, jnp.add)
        lanecum = _dotH(mf, lt_ll, ((1,), (0,)))
        rowoff = _dotH(lt_rr, rowcnt, ((1,), (0,)))
        p = jnp.where(mask, rowoff + lanecum, 300.0)
        pr = p.reshape(1, P)
        mhot = jnp.where(slot_iota == pr, 1.0, 0.0).astype(f32)
        vi3 = jnp.concatenate([s_row, flat_row, ones_row], axis=0)
        cc = _dotH(mhot, vi3, ((1,), (1,)))
        occ = cc[:, 2:3]
        vc = jnp.where(occ > 0.5, cc[:, 0:1], -1.0e9)
        ic = jnp.where(occ > 0.5, cc[:, 1:2], 1.0e9)
        vr = _dotH(vc, eye, ((0,), (0,)))
        ir = _dotH(ic, eye, ((0,), (0,)))
        bp = jnp.where((vr > vc) | ((vr == vc) & (ir < ic)),
                       1.0, 0.0).astype(f32)
        rank = _dotH(bp, ones_col, ((1,), (0,)))
        ohot = jnp.where(rank == out_iota_f, 1.0, 0.0).astype(f32)
        ti_row = _dotH(ir, ohot, ((1,), (0,)))
        return ti_row, tot

    def slow_select(smx):
        def extract(w):
            m = _allreduce_lanes(_allreduce_rows(w, jnp.maximum), jnp.maximum)
            cand = jnp.where(w == m, flat, big)
            pos = _allreduce_lanes(_allreduce_rows(cand, jnp.minimum),
                                   jnp.minimum)
            w = jnp.where(flat == pos, NEG_INF, w)
            return w, pos[:, 0:K]

        def rnd(r, carry):
            w, ti = carry
            w, pos = extract(w)
            ti = jnp.where(out_iota == r, pos, ti)
            return w, ti

        _, ti = lax.fori_loop(0, K, rnd, (smx, jnp.zeros((1, K), jnp.int32)))
        return ti

    ti_f, tot_t = fast_select(sm, s)
    bi_f, tot_b = fast_select(-sm, -s)
    ok = (jnp.max(tot_t) <= 128.0) & (jnp.max(tot_b) <= 128.0)

    @pl.when(ok)
    def _fast_write():
        ti_ref[...] = ti_f.astype(jnp.int32).reshape(1, 1, K)
        bi_ref[...] = bi_f.astype(jnp.int32).reshape(1, 1, K)

    @pl.when(jnp.logical_not(ok))
    def _slow_write():
        ti_ref[...] = slow_select(sm).reshape(1, 1, K)
        bi_ref[...] = slow_select(-sm).reshape(1, 1, K)


def _compute_topk_idx(ln, pn):
    return pl.pallas_call(
        _sim_topk_body,
        grid=(B,),
        in_specs=[
            pl.BlockSpec((1, P, D), lambda b: (b, 0, 0)),
            pl.BlockSpec((1, 1, D), lambda b: (b, 0, 0)),
        ],
        out_specs=[
            pl.BlockSpec((1, 1, K), lambda b: (b, 0, 0)),
            pl.BlockSpec((1, 1, K), lambda b: (b, 0, 0)),
        ],
        out_shape=[
            jax.ShapeDtypeStruct((B, 1, K), jnp.int32),
            jax.ShapeDtypeStruct((B, 1, K), jnp.int32),
        ],
    )(ln, pn)


def _make_gather_kernel():
    info = plsc.get_sparse_core_info()
    nc = info.num_cores
    mesh = plsc.VectorSubcoreMesh(core_axis_name="c", subcore_axis_name="s")

    @functools.partial(
        pl.kernel,
        out_type=(
            jax.ShapeDtypeStruct((B, K, D), jnp.float32),
            jax.ShapeDtypeStruct((B, K, D), jnp.float32),
        ),
        mesh=mesh,
        scratch_types=[
            pltpu.VMEM((K,), jnp.int32),
            pltpu.VMEM((K,), jnp.int32),
            pltpu.VMEM((16,), jnp.int32),
            pltpu.VMEM((K, D), jnp.float32),
            pltpu.SemaphoreType.DMA,
        ],
    )
    def gather_kernel(ti_hbm, bi_hbm, base_hbm, feat_hbm, tf_hbm, bf_hbm,
                      idxv, gidx, bv, rows, sem):
        w = lax.axis_index("s") * nc + lax.axis_index("c")
        for bi_ in range(B // 32):
            b = w * (B // 32) + bi_
            pltpu.sync_copy(base_hbm.at[b], bv)
            base16 = bv[...]
            for side in range(2):
                src = ti_hbm if side == 0 else bi_hbm
                dst = tf_hbm if side == 0 else bf_hbm
                pltpu.sync_copy(src.at[b], idxv)
                for j in range(K // 16):
                    gidx[pl.ds(j * 16, 16)] = idxv[pl.ds(j * 16, 16)] + base16
                pltpu.async_copy(feat_hbm.at[gidx], rows, sem).wait()
                pltpu.sync_copy(rows, dst.at[b])

    return gather_kernel


_gather_kernel = _make_gather_kernel()


def kernel(local_features, prototypes):
    pn = prototypes / (jnp.linalg.norm(prototypes, axis=-1, keepdims=True) + 1e-08)
    ti3, bi3 = _compute_topk_idx(local_features, pn)
    ti = ti3.reshape(B, K)
    bi = bi3.reshape(B, K)
    bases = jnp.broadcast_to((jnp.arange(B, dtype=jnp.int32) * P)[:, None], (B, 16))
    featrows = local_features.reshape(B * P, D)
    tf, bf = _gather_kernel(ti, bi, bases, featrows)
    return tf, bf, ti, bi
